# Initial kernel scaffold; baseline (speedup 1.0000x reference)
import functools

import jax
import jax.numpy as jnp
from jax import lax
from jax.experimental import pallas as pl
from jax.experimental.pallas import tpu as pltpu
from jax.experimental.pallas import tpu_sc as plsc

N = 10000
E = 320000
D = 128
EPS = 1e-5

NC = 2
NS = 16
NW = NC * NS
C = 80
CH = (E // C) // NW
ROWS_PER_TILE = N // NS


def _sc_mesh():
    return plsc.VectorSubcoreMesh(core_axis_name="c", subcore_axis_name="s")


def _sc_agg_body(with_deg, h_hbm, src_hbm, dst_hbm, z128_hbm, z16_hbm,
                 ones_hbm, agg_hbm, deg_hbm, sidx, didx, rows, ones_v,
                 agg_sh, deg_sh):
    cid = lax.axis_index("c")
    sid = lax.axis_index("s")
    wid = cid * NS + sid

    r0 = sid * ROWS_PER_TILE
    pltpu.sync_copy(z128_hbm, agg_sh.at[pl.ds(r0, ROWS_PER_TILE)])
    if with_deg:
        pltpu.sync_copy(z16_hbm, deg_sh.at[pl.ds(r0, ROWS_PER_TILE)])
        pltpu.sync_copy(ones_hbm, ones_v)

    cbase = wid * CH
    pltpu.sync_copy(src_hbm.at[pl.ds(cbase, CH)], sidx)
    pltpu.sync_copy(dst_hbm.at[pl.ds(cbase, CH)], didx)

    plsc.subcore_barrier()

    @pl.loop(0, CH)
    def _(k):
        pltpu.sync_copy(h_hbm.at[sidx.at[k]], rows)
        pltpu.sync_copy(rows, agg_sh.at[didx.at[k]], add=True)
        if with_deg:
            pltpu.sync_copy(ones_v, deg_sh.at[didx.at[k]], add=True)

    plsc.subcore_barrier()

    out_r = cid * N + r0
    pltpu.sync_copy(agg_sh.at[pl.ds(r0, ROWS_PER_TILE)],
                    agg_hbm.at[pl.ds(out_r, ROWS_PER_TILE)])
    if with_deg:
        pltpu.sync_copy(deg_sh.at[pl.ds(r0, ROWS_PER_TILE)],
                        deg_hbm.at[pl.ds(out_r, ROWS_PER_TILE)])


def _sc_agg(h, src2, dst2, z128, z16, ones, with_deg):
    out_type = [jax.ShapeDtypeStruct((NC * N, D), jnp.float32),
                jax.ShapeDtypeStruct((NC * N, 16), jnp.float32)]
    scratch = [
        pltpu.VMEM((CH, C), jnp.int32),
        pltpu.VMEM((CH, C), jnp.int32),
        pltpu.VMEM((C, D), jnp.float32),
        pltpu.VMEM((C, 16), jnp.float32),
        pltpu.VMEM_SHARED((N, D), jnp.float32),
        pltpu.VMEM_SHARED((N, 16), jnp.float32),
    ]
    fn = pl.kernel(functools.partial(_sc_agg_body, with_deg),
                   out_type=out_type, mesh=_sc_mesh(), scratch_types=scratch)
    return fn(h, src2, dst2, z128, z16, ones)


def _dense_body(has_base, args):
    if has_base:
        (aggp, degp, h, Wl, bl, Wr, g, b, base, out) = args
    else:
        (aggp, degp, h, Wl, bl, Wr, g, b, out) = args
    agg = aggp[0:N, :] + aggp[N:2 * N, :]
    deg = degp[0:N, 0:1] + degp[N:2 * N, 0:1]
    agg = agg / jnp.clip(deg, 1.0, None)
    t = (jnp.dot(agg, Wl[...], preferred_element_type=jnp.float32)
         + bl[...]
         + jnp.dot(h[...], Wr[...], preferred_element_type=jnp.float32))
    mu = jnp.mean(t, axis=-1, keepdims=True)
    var = jnp.mean((t - mu) ** 2, axis=-1, keepdims=True)
    y = (t - mu) * lax.rsqrt(var + EPS) * g[...] + b[...]
    y = jnp.maximum(y, 0.0)
    if has_base:
        y = y + base[...]
    out[...] = y


def _dense(aggp, degp, h, Wl, bl, Wr, g, b, base=None):
    inputs = [aggp, degp, h, Wl, bl.reshape(1, D), Wr,
              g.reshape(1, D), b.reshape(1, D)]
    if base is not None:
        inputs.append(base)
    body = lambda *args: _dense_body(base is not None, args)
    return pl.pallas_call(
        body,
        out_shape=jax.ShapeDtypeStruct((N, D), jnp.float32),
    )(*inputs)


def kernel(x, edge_index, Wl0, bl0, Wr0, g0, b0, Wl1, bl1, Wr1, g1, b1):
    src2 = edge_index[0].reshape(E // C, C)
    dst2 = edge_index[1].reshape(E // C, C)
    z128 = jnp.zeros((ROWS_PER_TILE, D), jnp.float32)
    z16 = jnp.zeros((ROWS_PER_TILE, 16), jnp.float32)
    ones = jnp.ones((C, 16), jnp.float32)

    aggp0, degp = _sc_agg(x, src2, dst2, z128, z16, ones, with_deg=True)
    h1 = _dense(aggp0, degp, x, Wl0, bl0, Wr0, g0, b0)
    aggp1, _ = _sc_agg(h1, src2, dst2, z128, z16, ones, with_deg=False)
    h2 = _dense(aggp1, degp, h1, Wl1, bl1, Wr1, g1, b1, base=x)
    return h2


# speedup vs baseline: 7.9580x; 7.9580x over previous
import dataclasses

import jax
import jax.numpy as jnp
from jax import lax
from jax.experimental import pallas as pl
from jax.experimental.pallas import tpu as pltpu
from jax.experimental.pallas import tpu_sc as plsc

N = 10000
E = 320000
D = 128
EPS = 1e-5

NC = 2
NS = 16
NW = NC * NS
C = 80
CH = (E // C) // NW
NP = 10240
RPT = NP // NS


def _sc_mesh():
    return plsc.VectorSubcoreMesh(core_axis_name="c", subcore_axis_name="s")


def _sc_agg_body(h_hbm, src_hbm, dst_hbm, z128_hbm, agg_hbm,
                 sidx, didx, rows, agg_sh):
    cid = lax.axis_index("c")
    sid = lax.axis_index("s")
    wid = cid * NS + sid

    r0 = sid * RPT
    pltpu.sync_copy(z128_hbm, agg_sh.at[pl.ds(r0, RPT)])

    pltpu.sync_copy(src_hbm.at[wid], sidx)
    pltpu.sync_copy(dst_hbm.at[wid], didx)

    plsc.subcore_barrier()

    @pl.loop(0, CH)
    def _(k):
        pltpu.sync_copy(h_hbm.at[sidx.at[k]], rows)
        pltpu.sync_copy(rows, agg_sh.at[didx.at[k]], add=True)

    plsc.subcore_barrier()

    pltpu.sync_copy(agg_sh.at[pl.ds(r0, RPT)], agg_hbm.at[wid])


def _sc_agg(h, src3, dst3, z128):
    out_type = jax.ShapeDtypeStruct((NW, RPT, D), jnp.float32)
    scratch = [
        pltpu.VMEM((CH, C), jnp.int32),
        pltpu.VMEM((CH, C), jnp.int32),
        pltpu.VMEM((C, D), jnp.float32),
        pltpu.VMEM_SHARED((NP, D), jnp.float32),
    ]
    fn = pl.kernel(_sc_agg_body, out_type=out_type, mesh=_sc_mesh(),
                   scratch_types=scratch)
    return fn(h, src3, dst3, z128).reshape(NC * NP, D)


def _sc_deg_body(dst_hbm, deg_hbm, didx, deg_l):
    cid = lax.axis_index("c")
    sid = lax.axis_index("s")
    wid = cid * NS + sid

    pltpu.sync_copy(dst_hbm.at[wid], didx)

    @pl.loop(0, NP, step=16)
    def _(i):
        deg_l[pl.ds(i, 16)] = jnp.zeros((16,), jnp.float32)

    ones_v = jnp.full((16,), 1.0, jnp.float32)

    @pl.loop(0, CH)
    def _(k):
        for l in range(C // 16):
            idx = didx[k, pl.ds(l * 16, 16)]
            plsc.addupdate_scatter(deg_l, [idx], ones_v)

    pltpu.sync_copy(deg_l, deg_hbm.at[wid])


def _sc_deg(dst3):
    out_type = jax.ShapeDtypeStruct((NW, NP), jnp.float32)
    scratch = [
        pltpu.VMEM((CH, C), jnp.int32),
        pltpu.VMEM((NP,), jnp.float32),
    ]
    cp = pltpu.CompilerParams()
    if "needs_layout_passes" in pltpu.CompilerParams.__dataclass_fields__:
        cp = dataclasses.replace(cp, needs_layout_passes=False)
    fn = pl.kernel(_sc_deg_body, out_type=out_type, mesh=_sc_mesh(),
                   scratch_types=scratch, compiler_params=cp)
    return fn(dst3)


def _dense_body(has_base, args):
    if has_base:
        (aggp, degp, h, Wl, bl, Wr, g, b, base, out) = args
    else:
        (aggp, degp, h, Wl, bl, Wr, g, b, out) = args
    agg = aggp[0:N, :] + aggp[NP:NP + N, :]
    deg = jnp.sum(degp[...], axis=0).reshape(NP, 1)[0:N]
    agg = agg / jnp.clip(deg, 1.0, None)
    t = (jnp.dot(agg, Wl[...], preferred_element_type=jnp.float32)
         + bl[...]
         + jnp.dot(h[...], Wr[...], preferred_element_type=jnp.float32))
    mu = jnp.mean(t, axis=-1, keepdims=True)
    var = jnp.mean((t - mu) ** 2, axis=-1, keepdims=True)
    y = (t - mu) * lax.rsqrt(var + EPS) * g[...] + b[...]
    y = jnp.maximum(y, 0.0)
    if has_base:
        y = y + base[...]
    out[...] = y


def _dense(aggp, degp, h, Wl, bl, Wr, g, b, base=None):
    inputs = [aggp, degp, h, Wl, bl.reshape(1, D), Wr,
              g.reshape(1, D), b.reshape(1, D)]
    if base is not None:
        inputs.append(base)
    body = lambda *args: _dense_body(base is not None, args)
    return pl.pallas_call(
        body,
        out_shape=jax.ShapeDtypeStruct((N, D), jnp.float32),
    )(*inputs)


def kernel(x, edge_index, Wl0, bl0, Wr0, g0, b0, Wl1, bl1, Wr1, g1, b1):
    src3 = edge_index[0].reshape(NW, CH, C)
    dst3 = edge_index[1].reshape(NW, CH, C)
    z128 = jnp.zeros((RPT, D), jnp.float32)

    degp = _sc_deg(dst3)
    aggp0 = _sc_agg(x, src3, dst3, z128)
    h1 = _dense(aggp0, degp, x, Wl0, bl0, Wr0, g0, b0)
    aggp1 = _sc_agg(h1, src3, dst3, z128)
    h2 = _dense(aggp1, degp, h1, Wl1, bl1, Wr1, g1, b1, base=x)
    return h2


# speedup vs baseline: 9.7604x; 1.2265x over previous
import dataclasses

import jax
import jax.numpy as jnp
from jax import lax
from jax.experimental import pallas as pl
from jax.experimental.pallas import tpu as pltpu
from jax.experimental.pallas import tpu_sc as plsc

N = 10000
E = 320000
D = 128
EPS = 1e-5

NC = 2
NS = 16
NW = NC * NS
C = 80
CH = (E // C) // NW
NB = 5
CB = CH // NB
NP = 10240
RPT = NP // NS


def _sc_mesh():
    return plsc.VectorSubcoreMesh(core_axis_name="c", subcore_axis_name="s")


def _sc_agg_body(h_hbm, src_hbm, dst_hbm, z128_hbm, agg_hbm,
                 sidx, didx, rowsbuf, sem0, sem1, agg_sh):
    rows0 = rowsbuf.at[pl.ds(0, C)]
    rows1 = rowsbuf.at[pl.ds(C, C)]
    cid = lax.axis_index("c")
    sid = lax.axis_index("s")
    wid = cid * NS + sid

    r0 = sid * RPT
    pltpu.sync_copy(z128_hbm, agg_sh.at[pl.ds(r0, RPT)])

    plsc.subcore_barrier()

    def gstart(k, buf, sem):
        pltpu.async_copy(h_hbm.at[sidx.at[k]], buf, sem)

    def gwait(k, buf, sem):
        pltpu.make_async_copy(h_hbm.at[sidx.at[k]], buf, sem).wait()

    def scat(k, buf):
        pltpu.sync_copy(buf, agg_sh.at[didx.at[k]], add=True)

    for blk in range(NB):
        pltpu.sync_copy(src_hbm.at[wid, blk], sidx)
        pltpu.sync_copy(dst_hbm.at[wid, blk], didx)
        gstart(0, rows0, sem0)

        @pl.loop(0, CB - 1, step=2)
        def _(k0):
            gwait(k0, rows0, sem0)
            gstart(k0 + 1, rows1, sem1)
            scat(k0, rows0)
            gwait(k0 + 1, rows1, sem1)
            gstart(k0 + 2, rows0, sem0)
            scat(k0 + 1, rows1)

        gwait(CB - 1, rows0, sem0)
        scat(CB - 1, rows0)

    plsc.subcore_barrier()

    pltpu.sync_copy(agg_sh.at[pl.ds(r0, RPT)], agg_hbm.at[wid])


def _sc_agg(h, src3, dst3, z128):
    out_type = jax.ShapeDtypeStruct((NW, RPT, D), jnp.float32)
    scratch = [
        pltpu.VMEM((CB, C), jnp.int32),
        pltpu.VMEM((CB, C), jnp.int32),
        pltpu.VMEM((2 * C, D), jnp.float32),
        pltpu.SemaphoreType.DMA,
        pltpu.SemaphoreType.DMA,
        pltpu.VMEM_SHARED((NP, D), jnp.float32),
    ]
    fn = pl.kernel(_sc_agg_body, out_type=out_type, mesh=_sc_mesh(),
                   scratch_types=scratch)
    return fn(h, src3, dst3, z128).reshape(NC * NP, D)


def _sc_deg_body(dst_hbm, deg_hbm, didx, deg_l):
    cid = lax.axis_index("c")
    sid = lax.axis_index("s")
    wid = cid * NS + sid

    pltpu.sync_copy(dst_hbm.at[wid], didx)

    @pl.loop(0, NP, step=16)
    def _(i):
        deg_l[pl.ds(i, 16)] = jnp.zeros((16,), jnp.float32)

    ones_v = jnp.full((16,), 1.0, jnp.float32)

    @pl.loop(0, CH)
    def _(k):
        for l in range(C // 16):
            idx = didx[k, pl.ds(l * 16, 16)]
            plsc.addupdate_scatter(deg_l, [idx], ones_v)

    pltpu.sync_copy(deg_l, deg_hbm.at[wid])


def _sc_deg(dst3):
    out_type = jax.ShapeDtypeStruct((NW, NP), jnp.float32)
    scratch = [
        pltpu.VMEM((CH, C), jnp.int32),
        pltpu.VMEM((NP,), jnp.float32),
    ]
    cp = pltpu.CompilerParams()
    if "needs_layout_passes" in pltpu.CompilerParams.__dataclass_fields__:
        cp = dataclasses.replace(cp, needs_layout_passes=False)
    fn = pl.kernel(_sc_deg_body, out_type=out_type, mesh=_sc_mesh(),
                   scratch_types=scratch, compiler_params=cp)
    return fn(dst3)


def _dense_body(has_base, args):
    if has_base:
        (aggp, degp, h, Wl, bl, Wr, g, b, base, out) = args
    else:
        (aggp, degp, h, Wl, bl, Wr, g, b, out) = args
    agg = aggp[0:N, :] + aggp[NP:NP + N, :]
    deg = jnp.sum(degp[...], axis=0).reshape(NP, 1)[0:N]
    agg = agg / jnp.clip(deg, 1.0, None)
    t = (jnp.dot(agg, Wl[...], preferred_element_type=jnp.float32)
         + bl[...]
         + jnp.dot(h[...], Wr[...], preferred_element_type=jnp.float32))
    mu = jnp.mean(t, axis=-1, keepdims=True)
    var = jnp.mean((t - mu) ** 2, axis=-1, keepdims=True)
    y = (t - mu) * lax.rsqrt(var + EPS) * g[...] + b[...]
    y = jnp.maximum(y, 0.0)
    if has_base:
        y = y + base[...]
    out[...] = y


def _dense(aggp, degp, h, Wl, bl, Wr, g, b, base=None):
    inputs = [aggp, degp, h, Wl, bl.reshape(1, D), Wr,
              g.reshape(1, D), b.reshape(1, D)]
    if base is not None:
        inputs.append(base)
    body = lambda *args: _dense_body(base is not None, args)
    return pl.pallas_call(
        body,
        out_shape=jax.ShapeDtypeStruct((N, D), jnp.float32),
    )(*inputs)


def kernel(x, edge_index, Wl0, bl0, Wr0, g0, b0, Wl1, bl1, Wr1, g1, b1):
    src4 = edge_index[0].reshape(NW, NB, CB, C)
    dst4 = edge_index[1].reshape(NW, NB, CB, C)
    dst3 = edge_index[1].reshape(NW, CH, C)
    z128 = jnp.zeros((RPT, D), jnp.float32)

    degp = _sc_deg(dst3)
    aggp0 = _sc_agg(x, src4, dst4, z128)
    h1 = _dense(aggp0, degp, x, Wl0, bl0, Wr0, g0, b0)
    aggp1 = _sc_agg(h1, src4, dst4, z128)
    h2 = _dense(aggp1, degp, h1, Wl1, bl1, Wr1, g1, b1, base=x)
    return h2


# speedup vs baseline: 10.9249x; 1.1193x over previous
import dataclasses

import jax
import jax.numpy as jnp
from jax import lax
from jax.experimental import pallas as pl
from jax.experimental.pallas import tpu as pltpu
from jax.experimental.pallas import tpu_sc as plsc

N = 10000
E = 320000
D = 128
EPS = 1e-5

NC = 2
NS = 16
NW = NC * NS
C = 100
CH = (E // C) // NW
NB = 4
CB = CH // NB
CDEG = 80
CHDEG = (E // CDEG) // NW
NP = 10240
RPT = NP // NS


def _sc_mesh():
    return plsc.VectorSubcoreMesh(core_axis_name="c", subcore_axis_name="s")


def _sc_agg_body(h_hbm, src_hbm, dst_hbm, z128_hbm, agg_hbm,
                 sA, dA, sB, dB, rowsbuf, semg0, semg1, semi, agg_sh):
    rb = (rowsbuf.at[0], rowsbuf.at[1])
    semg = (semg0, semg1)
    cid = lax.axis_index("c")
    sid = lax.axis_index("s")
    wid = cid * NS + sid

    r0 = sid * RPT
    pltpu.sync_copy(z128_hbm, agg_sh.at[pl.ds(r0, RPT)])

    plsc.subcore_barrier()

    def gstart(sbuf, k, b):
        pltpu.async_copy(h_hbm.at[sbuf.at[k]], rb[b], semg[b])

    def gwait(sbuf, k, b):
        pltpu.make_async_copy(h_hbm.at[sbuf.at[k]], rb[b], semg[b]).wait()

    def scat(dbuf, k, b):
        pltpu.sync_copy(rb[b], agg_sh.at[dbuf.at[k]], add=True)

    def istart(blk, sbuf, dbuf):
        pltpu.async_copy(src_hbm.at[wid, blk], sbuf, semi)
        pltpu.async_copy(dst_hbm.at[wid, blk], dbuf, semi)

    def iwait(blk, sbuf, dbuf):
        pltpu.make_async_copy(src_hbm.at[wid, blk], sbuf, semi).wait()
        pltpu.make_async_copy(dst_hbm.at[wid, blk], dbuf, semi).wait()

    pltpu.sync_copy(src_hbm.at[wid, 0], sA)
    pltpu.sync_copy(dst_hbm.at[wid, 0], dA)
    gstart(sA, 0, 0)
    for blk in range(NB):
        sbuf, dbuf = (sA, dA) if blk % 2 == 0 else (sB, dB)
        nsbuf, ndbuf = (sB, dB) if blk % 2 == 0 else (sA, dA)
        b0 = blk % 2
        b1 = 1 - b0
        if blk + 1 < NB:
            istart(blk + 1, nsbuf, ndbuf)

        @pl.loop(0, CB - 1, step=2)
        def _(k0):
            gwait(sbuf, k0, b0)
            gstart(sbuf, k0 + 1, b1)
            scat(dbuf, k0, b0)
            gwait(sbuf, k0 + 1, b1)
            gstart(sbuf, k0 + 2, b0)
            scat(dbuf, k0 + 1, b1)

        gwait(sbuf, CB - 1, b0)
        if blk + 1 < NB:
            iwait(blk + 1, nsbuf, ndbuf)
            gstart(nsbuf, 0, b1)
        scat(dbuf, CB - 1, b0)

    plsc.subcore_barrier()

    pltpu.sync_copy(agg_sh.at[pl.ds(r0, RPT)], agg_hbm.at[wid])


def _sc_agg(h, src3, dst3, z128):
    out_type = jax.ShapeDtypeStruct((NW, RPT, D), jnp.float32)
    scratch = [
        pltpu.VMEM((CB, C), jnp.int32),
        pltpu.VMEM((CB, C), jnp.int32),
        pltpu.VMEM((CB, C), jnp.int32),
        pltpu.VMEM((CB, C), jnp.int32),
        pltpu.VMEM((2, C, D), jnp.float32),
        pltpu.SemaphoreType.DMA,
        pltpu.SemaphoreType.DMA,
        pltpu.SemaphoreType.DMA,
        pltpu.VMEM_SHARED((NP, D), jnp.float32),
    ]
    fn = pl.kernel(_sc_agg_body, out_type=out_type, mesh=_sc_mesh(),
                   scratch_types=scratch)
    return fn(h, src3, dst3, z128).reshape(NC * NP, D)


def _sc_deg_body(dst_hbm, deg_hbm, didx, deg_l):
    cid = lax.axis_index("c")
    sid = lax.axis_index("s")
    wid = cid * NS + sid

    pltpu.sync_copy(dst_hbm.at[wid], didx)

    @pl.loop(0, NP, step=16)
    def _(i):
        deg_l[pl.ds(i, 16)] = jnp.zeros((16,), jnp.float32)

    ones_v = jnp.full((16,), 1.0, jnp.float32)

    @pl.loop(0, CHDEG)
    def _(k):
        for l in range(CDEG // 16):
            idx = didx[k, pl.ds(l * 16, 16)]
            plsc.addupdate_scatter(deg_l, [idx], ones_v)

    pltpu.sync_copy(deg_l, deg_hbm.at[wid])


def _sc_deg(dst3):
    out_type = jax.ShapeDtypeStruct((NW, NP), jnp.float32)
    scratch = [
        pltpu.VMEM((CHDEG, CDEG), jnp.int32),
        pltpu.VMEM((NP,), jnp.float32),
    ]
    cp = pltpu.CompilerParams()
    if "needs_layout_passes" in pltpu.CompilerParams.__dataclass_fields__:
        cp = dataclasses.replace(cp, needs_layout_passes=False)
    fn = pl.kernel(_sc_deg_body, out_type=out_type, mesh=_sc_mesh(),
                   scratch_types=scratch, compiler_params=cp)
    return fn(dst3)


def _dense_body(has_base, args):
    if has_base:
        (aggp, degp, h, Wl, bl, Wr, g, b, base, out) = args
    else:
        (aggp, degp, h, Wl, bl, Wr, g, b, out) = args
    agg = aggp[0:N, :] + aggp[NP:NP + N, :]
    deg = jnp.sum(degp[...], axis=0).reshape(NP, 1)[0:N]
    agg = agg / jnp.clip(deg, 1.0, None)
    t = (jnp.dot(agg, Wl[...], preferred_element_type=jnp.float32)
         + bl[...]
         + jnp.dot(h[...], Wr[...], preferred_element_type=jnp.float32))
    mu = jnp.mean(t, axis=-1, keepdims=True)
    var = jnp.mean((t - mu) ** 2, axis=-1, keepdims=True)
    y = (t - mu) * lax.rsqrt(var + EPS) * g[...] + b[...]
    y = jnp.maximum(y, 0.0)
    if has_base:
        y = y + base[...]
    out[...] = y


def _dense(aggp, degp, h, Wl, bl, Wr, g, b, base=None):
    inputs = [aggp, degp, h, Wl, bl.reshape(1, D), Wr,
              g.reshape(1, D), b.reshape(1, D)]
    if base is not None:
        inputs.append(base)
    body = lambda *args: _dense_body(base is not None, args)
    return pl.pallas_call(
        body,
        out_shape=jax.ShapeDtypeStruct((N, D), jnp.float32),
    )(*inputs)


def kernel(x, edge_index, Wl0, bl0, Wr0, g0, b0, Wl1, bl1, Wr1, g1, b1):
    src4 = edge_index[0].reshape(NW, NB, CB, C)
    dst4 = edge_index[1].reshape(NW, NB, CB, C)
    dst3 = edge_index[1].reshape(NW, CHDEG, CDEG)
    z128 = jnp.zeros((RPT, D), jnp.float32)

    degp = _sc_deg(dst3)
    aggp0 = _sc_agg(x, src4, dst4, z128)
    h1 = _dense(aggp0, degp, x, Wl0, bl0, Wr0, g0, b0)
    aggp1 = _sc_agg(h1, src4, dst4, z128)
    h2 = _dense(aggp1, degp, h1, Wl1, bl1, Wr1, g1, b1, base=x)
    return h2
